# Initial kernel scaffold; baseline (speedup 1.0000x reference)
import jax
import jax.numpy as jnp
from jax.experimental import pallas as pl


def kernel(x, edge_weight, vertex, edges, batch, w_n2l, w_e2l, p_node_conv, trans_node_1, trans_node_2, h1_weight, h2_weight):
    raise NotImplementedError("write your pallas kernel here")


# speedup vs baseline: 1.0800x; 1.0800x over previous
import functools

import jax
import jax.numpy as jnp
from jax import lax
from jax.experimental import pallas as pl
from jax.experimental.pallas import tpu as pltpu
from jax.experimental.pallas import tpu_sc as plsc

_N = 10000
_NNZ = 320000
_NHE = 40000
_G = 64
_E = 128
_T = 3

_NP = 10240
_NNZP = 327680
_CS = 10240
_XEP = 4 * _CS
_DUMMY = _N
_B = 128
_NS = 16
_ROWS_PER_TILE = _NP // _NS

_mesh = plsc.VectorSubcoreMesh(core_axis_name="c", subcore_axis_name="s")


@functools.partial(
    pl.kernel,
    out_type=jax.ShapeDtypeStruct((_XEP, _E), jnp.float32),
    mesh=_mesh,
    scratch_types=[
        pltpu.VMEM((_B,), jnp.int32),
        pltpu.VMEM((_B,), jnp.int32),
        pltpu.VMEM((_B,), jnp.int32),
        pltpu.VMEM((_B, _E), jnp.float32),
        pltpu.VMEM_SHARED((_NP, _E), jnp.float32),
        pltpu.SemaphoreType.DMA,
    ],
)
def _sc_edge_sum(h_hbm, vert_hbm, edge_hbm, zeros_hbm, xe_hbm,
                 vblk, eblk, sidx, rows, acc, sem):
    c = lax.axis_index("c")
    s = lax.axis_index("s")
    pairs_per_tile = _NNZP // _NS
    nblk = pairs_per_tile // _B
    for r in range(2):
        base = (2 * c + r) * _CS
        pltpu.sync_copy(zeros_hbm, acc.at[pl.ds(s * _ROWS_PER_TILE, _ROWS_PER_TILE)])
        plsc.subcore_barrier()

        def body(i, carry):
            p0 = s * pairs_per_tile + i * _B
            pltpu.sync_copy(vert_hbm.at[pl.ds(p0, _B)], vblk)
            pltpu.sync_copy(edge_hbm.at[pl.ds(p0, _B)], eblk)
            for j in range(_B // 16):
                e16 = eblk[pl.ds(j * 16, 16)]
                rel = e16 - base
                ok = (rel >= 0) & (rel < _N)
                sidx[pl.ds(j * 16, 16)] = jnp.where(ok, rel, _DUMMY)
            pltpu.async_copy(h_hbm.at[vblk], rows, sem).wait()
            pltpu.sync_copy(rows, acc.at[sidx], add=True)
            return carry

        lax.fori_loop(0, nblk, body, 0)
        plsc.subcore_barrier()
        pltpu.sync_copy(
            acc.at[pl.ds(s * _ROWS_PER_TILE, _ROWS_PER_TILE)],
            xe_hbm.at[pl.ds(base + s * _ROWS_PER_TILE, _ROWS_PER_TILE)])
        plsc.subcore_barrier()


def _make_vertex_sum(gather_rows: bool, src_rows: int):
    @functools.partial(
        pl.kernel,
        out_type=jax.ShapeDtypeStruct((2 * _NP, _E), jnp.float32),
        mesh=_mesh,
        scratch_types=[
            pltpu.VMEM((_B,), jnp.int32),
            pltpu.VMEM((_B,), jnp.int32),
            pltpu.VMEM((_B, _E), jnp.float32),
            pltpu.VMEM_SHARED((_NP, _E), jnp.float32),
            pltpu.SemaphoreType.DMA,
        ],
    )
    def _sc_vertex_sum(src_hbm, vert_hbm, edge_hbm, zeros_hbm, xv_hbm,
                       vblk, eblk, rows, acc, sem):
        c = lax.axis_index("c")
        s = lax.axis_index("s")
        pairs_per_tile = _NNZP // (2 * _NS)
        nblk = pairs_per_tile // _B
        pltpu.sync_copy(zeros_hbm, acc.at[pl.ds(s * _ROWS_PER_TILE, _ROWS_PER_TILE)])
        plsc.subcore_barrier()

        def body(i, carry):
            p0 = (c * _NS + s) * pairs_per_tile + i * _B
            pltpu.sync_copy(vert_hbm.at[pl.ds(p0, _B)], vblk)
            if gather_rows:
                pltpu.sync_copy(edge_hbm.at[pl.ds(p0, _B)], eblk)
                pltpu.async_copy(src_hbm.at[eblk], rows, sem).wait()
            else:
                pltpu.sync_copy(src_hbm.at[pl.ds(p0, _B)], rows)
            pltpu.sync_copy(rows, acc.at[vblk], add=True)
            return carry

        lax.fori_loop(0, nblk, body, 0)
        plsc.subcore_barrier()
        pltpu.sync_copy(
            acc.at[pl.ds(s * _ROWS_PER_TILE, _ROWS_PER_TILE)],
            xv_hbm.at[pl.ds(c * _NP + s * _ROWS_PER_TILE, _ROWS_PER_TILE)])

    return _sc_vertex_sum


_sc_vertex_sum_gather = _make_vertex_sum(True, _XEP)
_sc_vertex_sum_linear = _make_vertex_sum(False, _NNZP)


_BR = 1024
_NB = _NP // _BR
_BRE = 5120
_NBE = _NNZP // _BRE


def _tc_prologue(x_p, w_n2l, t2):
    def body(x_ref, w_ref, t2_ref, h0_ref, sa_ref):
        xb = x_ref[...]
        h0_ref[...] = jnp.maximum(
            jnp.dot(xb, w_ref[...], preferred_element_type=jnp.float32), 0.0)
        sa_ref[...] = xb[:, 1:2] * t2_ref[...]
    return pl.pallas_call(
        body,
        grid=(_NB,),
        in_specs=[
            pl.BlockSpec((_BR, 2), lambda i: (i, 0)),
            pl.BlockSpec((2, _E), lambda i: (0, 0)),
            pl.BlockSpec((1, _E), lambda i: (0, 0)),
        ],
        out_specs=[
            pl.BlockSpec((_BR, _E), lambda i: (i, 0)),
            pl.BlockSpec((_BR, _E), lambda i: (i, 0)),
        ],
        out_shape=[
            jax.ShapeDtypeStruct((_NP, _E), jnp.float32),
            jax.ShapeDtypeStruct((_NP, _E), jnp.float32),
        ],
    )(x_p, w_n2l, t2)


def _tc_edge_feat(ew_p, w_e2l):
    def body(ew_ref, w_ref, out_ref):
        out_ref[...] = jnp.maximum(
            jnp.dot(ew_ref[...], w_ref[...], preferred_element_type=jnp.float32), 0.0)
    return pl.pallas_call(
        body,
        grid=(_NBE,),
        in_specs=[
            pl.BlockSpec((_BRE, 4), lambda i: (i, 0)),
            pl.BlockSpec((4, _E), lambda i: (0, 0)),
        ],
        out_specs=pl.BlockSpec((_BRE, _E), lambda i: (i, 0)),
        out_shape=jax.ShapeDtypeStruct((_NNZP, _E), jnp.float32),
    )(ew_p, w_e2l)


def _tc_combine(a, b, w, add, relu):
    def body(a_ref, b_ref, w_ref, add_ref, out_ref):
        acc = jnp.dot(a_ref[...] + b_ref[...], w_ref[...],
                      preferred_element_type=jnp.float32) + add_ref[...]
        out_ref[...] = jnp.maximum(acc, 0.0) if relu else acc
    return pl.pallas_call(
        body,
        grid=(_NB,),
        in_specs=[
            pl.BlockSpec((_BR, _E), lambda i: (i, 0)),
            pl.BlockSpec((_BR, _E), lambda i: (i, 0)),
            pl.BlockSpec((_E, _E), lambda i: (0, 0)),
            pl.BlockSpec((_BR, _E), lambda i: (i, 0)),
        ],
        out_specs=pl.BlockSpec((_BR, _E), lambda i: (i, 0)),
        out_shape=jax.ShapeDtypeStruct((_NP, _E), jnp.float32),
    )(a, b, w, add)


def _tc_graph_pool(h, batch3):
    def body(h_ref, b_ref, y_ref):
        i = pl.program_id(0)
        @pl.when(i == 0)
        def _():
            y_ref[...] = jnp.zeros_like(y_ref)
        b = b_ref[0, 0, :]
        m = (b[:, None] == lax.broadcasted_iota(jnp.int32, (_BR, _G), 1)
             ).astype(jnp.float32)
        y_ref[...] += lax.dot_general(
            m, h_ref[...], (((0,), (0,)), ((), ())),
            preferred_element_type=jnp.float32)
    return pl.pallas_call(
        body,
        grid=(_NB,),
        in_specs=[
            pl.BlockSpec((_BR, _E), lambda i: (i, 0)),
            pl.BlockSpec((1, 1, _BR), lambda i: (i, 0, 0)),
        ],
        out_specs=pl.BlockSpec((_G, _E), lambda i: (0, 0)),
        out_shape=jax.ShapeDtypeStruct((_G, _E), jnp.float32),
    )(h, batch3)


def _tc_head(h, batch3, y, h1t, h1b, h2):
    def body(h_ref, b_ref, y_ref, h1t_ref, h1b_ref, h2_ref, q_ref):
        z = jnp.dot(y_ref[...], h1b_ref[...], preferred_element_type=jnp.float32)
        b = b_ref[0, 0, :]
        m = (b[:, None] == lax.broadcasted_iota(jnp.int32, (_BR, _G), 1)
             ).astype(jnp.float32)
        hid = jnp.maximum(
            jnp.dot(h_ref[...], h1t_ref[...], preferred_element_type=jnp.float32)
            + jnp.dot(m, z, preferred_element_type=jnp.float32), 0.0)
        q_ref[...] = jnp.dot(hid, h2_ref[...], preferred_element_type=jnp.float32)
    return pl.pallas_call(
        body,
        grid=(_NB,),
        in_specs=[
            pl.BlockSpec((_BR, _E), lambda i: (i, 0)),
            pl.BlockSpec((1, 1, _BR), lambda i: (i, 0, 0)),
            pl.BlockSpec((_G, _E), lambda i: (0, 0)),
            pl.BlockSpec((_E, _G), lambda i: (0, 0)),
            pl.BlockSpec((_E, _G), lambda i: (0, 0)),
            pl.BlockSpec((_G, 1), lambda i: (0, 0)),
        ],
        out_specs=pl.BlockSpec((_BR, 1), lambda i: (i, 0)),
        out_shape=jax.ShapeDtypeStruct((_NP, 1), jnp.float32),
    )(h, batch3, y, h1t, h1b, h2)


def kernel(x, edge_weight, vertex, edges, batch, w_n2l, w_e2l, p_node_conv,
           trans_node_1, trans_node_2, h1_weight, h2_weight):
    x_p = jnp.pad(x, ((0, _NP - _N), (0, 0)))
    ew_p = jnp.pad(edge_weight, ((0, _NNZP - _NNZ), (0, 0)))
    vert_p = jnp.pad(vertex.astype(jnp.int32), (0, _NNZP - _NNZ),
                     constant_values=_DUMMY)
    e32 = edges.astype(jnp.int32)
    edge_p = jnp.pad(e32 + (_CS - _N) * (e32 // _N), (0, _NNZP - _NNZ),
                     constant_values=_N)
    batch3 = jnp.pad(batch.astype(jnp.int32), (0, _NP - _N),
                     constant_values=_G).reshape(_NB, 1, _BR)
    zeros_hbm = jnp.zeros((_ROWS_PER_TILE, _E), jnp.float32)
    h1t, h1b = h1_weight[:_E], h1_weight[_E:]

    h0, sattr = _tc_prologue(x_p, w_n2l, trans_node_2)
    ea = _tc_edge_feat(ew_p, w_e2l)
    p2 = _sc_vertex_sum_linear(ea, vert_p, edge_p, zeros_hbm)
    bias = _tc_combine(p2[:_NP], p2[_NP:], trans_node_1, sattr, relu=False)

    h = h0
    for _ in range(_T):
        xe = _sc_edge_sum(h, vert_p, edge_p, zeros_hbm)
        xv = _sc_vertex_sum_gather(xe, vert_p, edge_p, zeros_hbm)
        h = _tc_combine(xv[:_NP], xv[_NP:], p_node_conv, bias, relu=True)

    y = _tc_graph_pool(h, batch3)
    q = _tc_head(h, batch3, y, h1t, h1b, h2_weight)
    return q[:_N]


# speedup vs baseline: 1.2825x; 1.1874x over previous
import functools

import jax
import jax.numpy as jnp
from jax import lax
from jax.experimental import pallas as pl
from jax.experimental.pallas import tpu as pltpu
from jax.experimental.pallas import tpu_sc as plsc

_N = 10000
_NNZ = 320000
_NHE = 40000
_G = 64
_E = 128
_T = 3

_NP = 10240
_NNZP = 327680
_CS = 10240
_XEP = 4 * _CS
_DUMMY = _N
_B = 128
_NS = 16
_ROWS_PER_TILE = _NP // _NS

_mesh = plsc.VectorSubcoreMesh(core_axis_name="c", subcore_axis_name="s")


_D = 2
_SB = 2048
_BPS = _SB // _B


def _superblock(stage_idx, start_gather, wait_gather, scatter_block):
    stage_idx()
    for k in range(_BPS + _D):
        d = k % _D
        if k >= _D:
            wait_gather(k - _D, d)
            scatter_block(k - _D, d)
        if k < _BPS:
            start_gather(k, d)


@functools.partial(
    pl.kernel,
    out_type=jax.ShapeDtypeStruct((_XEP, _E), jnp.float32),
    mesh=_mesh,
    scratch_types=[
        pltpu.VMEM((_SB,), jnp.int32),
        pltpu.VMEM((_BPS, _B), jnp.int32),
        [pltpu.VMEM((_B, _E), jnp.float32) for _ in range(_D)],
        pltpu.VMEM_SHARED((_NP, _E), jnp.float32),
        [pltpu.SemaphoreType.DMA for _ in range(_D)],
    ],
)
def _sc_edge_sum(h_hbm, vert_hbm, rel_hbm, zeros_hbm, xe_hbm,
                 gidx, sidx, rows, acc, sems):
    c = lax.axis_index("c")
    s = lax.axis_index("s")
    ppt = _NNZP // _NS
    nsb = ppt // _SB
    for r in range(2):
        chunk = 2 * c + r
        base = chunk * _CS
        pltpu.sync_copy(zeros_hbm, acc.at[pl.ds(s * _ROWS_PER_TILE, _ROWS_PER_TILE)])
        plsc.subcore_barrier()

        def sb_body(i, carry):
            pr0 = pl.multiple_of(s * ppt + i * _SB, _SB)
            ir0 = pl.multiple_of(chunk * (_NNZP // _B) + (pr0 // _B), _BPS)

            def stage_idx():
                pltpu.sync_copy(vert_hbm.at[pl.ds(pr0, _SB)], gidx)
                pltpu.sync_copy(rel_hbm.at[pl.ds(ir0, _BPS)], sidx)

            def start_gather(k, d):
                pltpu.async_copy(h_hbm.at[gidx.at[pl.ds(k * _B, _B)]],
                                 rows[d], sems[d])

            def wait_gather(k, d):
                pltpu.make_async_copy(h_hbm.at[gidx.at[pl.ds(k * _B, _B)]],
                                      rows[d], sems[d]).wait()

            def scatter_block(k, d):
                pltpu.sync_copy(rows[d], acc.at[sidx.at[k]], add=True)

            _superblock(stage_idx, start_gather, wait_gather, scatter_block)
            return carry

        lax.fori_loop(0, nsb, sb_body, 0)
        plsc.subcore_barrier()
        pltpu.sync_copy(
            acc.at[pl.ds(s * _ROWS_PER_TILE, _ROWS_PER_TILE)],
            xe_hbm.at[pl.ds(base + s * _ROWS_PER_TILE, _ROWS_PER_TILE)])
        plsc.subcore_barrier()


def _make_vertex_sum(gather_rows: bool):
    ppt = _NNZP // (2 * _NS)
    nsb = ppt // _SB

    @functools.partial(
        pl.kernel,
        out_type=jax.ShapeDtypeStruct((2 * _NP, _E), jnp.float32),
        mesh=_mesh,
        scratch_types=[
            pltpu.VMEM((_SB,), jnp.int32),
            pltpu.VMEM((_BPS, _B), jnp.int32),
            [pltpu.VMEM((_B, _E), jnp.float32) for _ in range(_D)],
            pltpu.VMEM_SHARED((_NP, _E), jnp.float32),
            [pltpu.SemaphoreType.DMA for _ in range(_D)],
        ],
    )
    def _sc_vertex_sum(src_hbm, vert2_hbm, edge_hbm, zeros_hbm, xv_hbm,
                       gidx, sidx, rows, acc, sems):
        c = lax.axis_index("c")
        s = lax.axis_index("s")
        w = c * _NS + s
        pltpu.sync_copy(zeros_hbm, acc.at[pl.ds(s * _ROWS_PER_TILE, _ROWS_PER_TILE)])
        plsc.subcore_barrier()

        def sb_body(i, carry):
            pr0 = pl.multiple_of(w * ppt + i * _SB, _SB)

            def stage_idx():
                if gather_rows:
                    pltpu.sync_copy(edge_hbm.at[pl.ds(pr0, _SB)], gidx)
                pltpu.sync_copy(
                    vert2_hbm.at[pl.ds(pl.multiple_of(pr0 // _B, _BPS), _BPS)],
                    sidx)

            if gather_rows:
                def start_gather(k, d):
                    pltpu.async_copy(src_hbm.at[gidx.at[pl.ds(k * _B, _B)]],
                                     rows[d], sems[d])

                def wait_gather(k, d):
                    pltpu.make_async_copy(src_hbm.at[gidx.at[pl.ds(k * _B, _B)]],
                                          rows[d], sems[d]).wait()
            else:
                def start_gather(k, d):
                    pltpu.async_copy(src_hbm.at[pl.ds(pr0 + k * _B, _B)],
                                     rows[d], sems[d])

                def wait_gather(k, d):
                    pltpu.make_async_copy(src_hbm.at[pl.ds(pr0 + k * _B, _B)],
                                          rows[d], sems[d]).wait()

            def scatter_block(k, d):
                pltpu.sync_copy(rows[d], acc.at[sidx.at[k]], add=True)

            _superblock(stage_idx, start_gather, wait_gather, scatter_block)
            return carry

        lax.fori_loop(0, nsb, sb_body, 0)
        plsc.subcore_barrier()
        pltpu.sync_copy(
            acc.at[pl.ds(s * _ROWS_PER_TILE, _ROWS_PER_TILE)],
            xv_hbm.at[pl.ds(c * _NP + s * _ROWS_PER_TILE, _ROWS_PER_TILE)])

    return _sc_vertex_sum


_sc_vertex_sum_gather = _make_vertex_sum(True)
_sc_vertex_sum_linear = _make_vertex_sum(False)


_BR = 1024
_NB = _NP // _BR
_BRE = 5120
_NBE = _NNZP // _BRE


def _tc_prologue(x_p, w_n2l, t2):
    def body(x_ref, w_ref, t2_ref, h0_ref, sa_ref):
        xb = x_ref[...]
        h0_ref[...] = jnp.maximum(
            jnp.dot(xb, w_ref[...], preferred_element_type=jnp.float32), 0.0)
        sa_ref[...] = xb[:, 1:2] * t2_ref[...]
    return pl.pallas_call(
        body,
        grid=(_NB,),
        in_specs=[
            pl.BlockSpec((_BR, 2), lambda i: (i, 0)),
            pl.BlockSpec((2, _E), lambda i: (0, 0)),
            pl.BlockSpec((1, _E), lambda i: (0, 0)),
        ],
        out_specs=[
            pl.BlockSpec((_BR, _E), lambda i: (i, 0)),
            pl.BlockSpec((_BR, _E), lambda i: (i, 0)),
        ],
        out_shape=[
            jax.ShapeDtypeStruct((_NP, _E), jnp.float32),
            jax.ShapeDtypeStruct((_NP, _E), jnp.float32),
        ],
    )(x_p, w_n2l, t2)


def _tc_edge_feat(ew_p, w_e2l):
    def body(ew_ref, w_ref, out_ref):
        out_ref[...] = jnp.maximum(
            jnp.dot(ew_ref[...], w_ref[...], preferred_element_type=jnp.float32), 0.0)
    return pl.pallas_call(
        body,
        grid=(_NBE,),
        in_specs=[
            pl.BlockSpec((_BRE, 4), lambda i: (i, 0)),
            pl.BlockSpec((4, _E), lambda i: (0, 0)),
        ],
        out_specs=pl.BlockSpec((_BRE, _E), lambda i: (i, 0)),
        out_shape=jax.ShapeDtypeStruct((_NNZP, _E), jnp.float32),
    )(ew_p, w_e2l)


def _tc_combine(a, b, w, add, relu):
    def body(a_ref, b_ref, w_ref, add_ref, out_ref):
        acc = jnp.dot(a_ref[...] + b_ref[...], w_ref[...],
                      preferred_element_type=jnp.float32) + add_ref[...]
        out_ref[...] = jnp.maximum(acc, 0.0) if relu else acc
    return pl.pallas_call(
        body,
        grid=(_NB,),
        in_specs=[
            pl.BlockSpec((_BR, _E), lambda i: (i, 0)),
            pl.BlockSpec((_BR, _E), lambda i: (i, 0)),
            pl.BlockSpec((_E, _E), lambda i: (0, 0)),
            pl.BlockSpec((_BR, _E), lambda i: (i, 0)),
        ],
        out_specs=pl.BlockSpec((_BR, _E), lambda i: (i, 0)),
        out_shape=jax.ShapeDtypeStruct((_NP, _E), jnp.float32),
    )(a, b, w, add)


def _tc_graph_pool(h, batch3):
    def body(h_ref, b_ref, y_ref):
        i = pl.program_id(0)
        @pl.when(i == 0)
        def _():
            y_ref[...] = jnp.zeros_like(y_ref)
        b = b_ref[0, 0, :]
        m = (b[:, None] == lax.broadcasted_iota(jnp.int32, (_BR, _G), 1)
             ).astype(jnp.float32)
        y_ref[...] += lax.dot_general(
            m, h_ref[...], (((0,), (0,)), ((), ())),
            preferred_element_type=jnp.float32)
    return pl.pallas_call(
        body,
        grid=(_NB,),
        in_specs=[
            pl.BlockSpec((_BR, _E), lambda i: (i, 0)),
            pl.BlockSpec((1, 1, _BR), lambda i: (i, 0, 0)),
        ],
        out_specs=pl.BlockSpec((_G, _E), lambda i: (0, 0)),
        out_shape=jax.ShapeDtypeStruct((_G, _E), jnp.float32),
    )(h, batch3)


def _tc_head(h, batch3, y, h1t, h1b, h2):
    def body(h_ref, b_ref, y_ref, h1t_ref, h1b_ref, h2_ref, q_ref):
        z = jnp.dot(y_ref[...], h1b_ref[...], preferred_element_type=jnp.float32)
        b = b_ref[0, 0, :]
        m = (b[:, None] == lax.broadcasted_iota(jnp.int32, (_BR, _G), 1)
             ).astype(jnp.float32)
        hid = jnp.maximum(
            jnp.dot(h_ref[...], h1t_ref[...], preferred_element_type=jnp.float32)
            + jnp.dot(m, z, preferred_element_type=jnp.float32), 0.0)
        q_ref[...] = jnp.dot(hid, h2_ref[...], preferred_element_type=jnp.float32)
    return pl.pallas_call(
        body,
        grid=(_NB,),
        in_specs=[
            pl.BlockSpec((_BR, _E), lambda i: (i, 0)),
            pl.BlockSpec((1, 1, _BR), lambda i: (i, 0, 0)),
            pl.BlockSpec((_G, _E), lambda i: (0, 0)),
            pl.BlockSpec((_E, _G), lambda i: (0, 0)),
            pl.BlockSpec((_E, _G), lambda i: (0, 0)),
            pl.BlockSpec((_G, 1), lambda i: (0, 0)),
        ],
        out_specs=pl.BlockSpec((_BR, 1), lambda i: (i, 0)),
        out_shape=jax.ShapeDtypeStruct((_NP, 1), jnp.float32),
    )(h, batch3, y, h1t, h1b, h2)


def kernel(x, edge_weight, vertex, edges, batch, w_n2l, w_e2l, p_node_conv,
           trans_node_1, trans_node_2, h1_weight, h2_weight):
    x_p = jnp.pad(x, ((0, _NP - _N), (0, 0)))
    ew_p = jnp.pad(edge_weight, ((0, _NNZP - _NNZ), (0, 0)))
    vert_p = jnp.pad(vertex.astype(jnp.int32), (0, _NNZP - _NNZ),
                     constant_values=_DUMMY)
    vert2 = vert_p.reshape(_NNZP // _B, _B)
    e32 = edges.astype(jnp.int32)
    edge_p = jnp.pad(e32 + (_CS - _N) * (e32 // _N), (0, _NNZP - _NNZ),
                     constant_values=_N)
    echunk = edge_p // _CS
    erel = edge_p - echunk * _CS
    rel_all = jnp.stack([jnp.where(echunk == cc, erel, _DUMMY)
                         for cc in range(4)]).reshape(4 * (_NNZP // _B), _B)
    batch3 = jnp.pad(batch.astype(jnp.int32), (0, _NP - _N),
                     constant_values=_G).reshape(_NB, 1, _BR)
    zeros_hbm = jnp.zeros((_ROWS_PER_TILE, _E), jnp.float32)
    h1t, h1b = h1_weight[:_E], h1_weight[_E:]

    h0, sattr = _tc_prologue(x_p, w_n2l, trans_node_2)
    ea = _tc_edge_feat(ew_p, w_e2l)
    p2 = _sc_vertex_sum_linear(ea, vert2, edge_p, zeros_hbm)
    bias = _tc_combine(p2[:_NP], p2[_NP:], trans_node_1, sattr, relu=False)

    h = h0
    for _ in range(_T):
        xe = _sc_edge_sum(h, vert_p, rel_all, zeros_hbm)
        xv = _sc_vertex_sum_gather(xe, vert2, edge_p, zeros_hbm)
        h = _tc_combine(xv[:_NP], xv[_NP:], p_node_conv, bias, relu=True)

    y = _tc_graph_pool(h, batch3)
    q = _tc_head(h, batch3, y, h1t, h1b, h2_weight)
    return q[:_N]
